# baseline (speedup 1.0000x reference)
import functools

import jax
import jax.numpy as jnp
from jax import lax
from jax.experimental import pallas as pl
from jax.experimental.pallas import tpu as pltpu
from jax.experimental.pallas import tpu_sc as plsc

_NC, _NS, _L = 2, 16, 16
_NW = _NC * _NS
_B = 4
_S = 8192
_D = 1024
_ROWS = _S // _NW
_CH = 4
_NCH = _ROWS // _CH


def _sc_body(mask_hbm, table_hbm, out_hbm, mask_v, tbuf, obuf, sem_t, sem_o):
    wid = lax.axis_index("s") * _NC + lax.axis_index("c")
    base = wid * _ROWS

    for b in range(_B):
        pltpu.sync_copy(
            mask_hbm.at[b, pl.ds(base, _ROWS)], mask_v.at[pl.ds(b * _ROWS, _ROWS)]
        )

    pltpu.async_copy(table_hbm.at[pl.ds(base, _CH)], tbuf.at[0], sem_t)

    def compute_chunk(c, k):
        crow = c * _CH

        nxt = jnp.minimum(c + 1, _NCH - 1) * _CH
        pltpu.async_copy(table_hbm.at[pl.ds(base + nxt, _CH)], tbuf.at[(k + 1) % 3], sem_t)

        pltpu.make_async_copy(
            table_hbm.at[pl.ds(base + crow, _CH)], tbuf.at[k], sem_t
        ).wait()

        @pl.when(c >= 3)
        def _():
            pltpu.make_async_copy(
                obuf.at[k], out_hbm.at[:, pl.ds(base, _CH)], sem_o
            ).wait()

        @plsc.parallel_loop(0, _CH)
        def _row(r):
            mb = [
                jnp.broadcast_to(
                    mask_v[pl.ds(b * _ROWS + crow + r, _L)][0], (_L,)
                )
                for b in range(_B)
            ]
            for dch in range(_D // _L):
                tv = tbuf[k, r, pl.ds(dch * _L, _L)]
                for b in range(_B):
                    obuf[k, b, r, pl.ds(dch * _L, _L)] = tv * mb[b]

        pltpu.async_copy(
            obuf.at[k], out_hbm.at[:, pl.ds(base + crow, _CH)], sem_o
        )

    def outer(g, carry):
        for j in range(3):
            compute_chunk(g * 3 + j, j)
        return carry

    lax.fori_loop(0, _NCH // 3, outer, 0, unroll=False)

    for c in range(_NCH - _NCH % 3, _NCH):
        compute_chunk(c, c % 3)

    pltpu.make_async_copy(
        table_hbm.at[pl.ds(base, _CH)], tbuf.at[0], sem_t
    ).wait()
    for k in range(3):
        pltpu.make_async_copy(
            obuf.at[k], out_hbm.at[:, pl.ds(base, _CH)], sem_o
        ).wait()


@functools.partial(
    pl.kernel,
    out_type=jax.ShapeDtypeStruct((_B, _S, _D), jnp.float32),
    mesh=plsc.VectorSubcoreMesh(core_axis_name="c", subcore_axis_name="s"),
    scratch_types=[
        pltpu.VMEM((_B * _ROWS + _L,), jnp.float32),
        pltpu.VMEM((3, _CH, _D), jnp.float32),
        pltpu.VMEM((3, _B, _CH, _D), jnp.float32),
        pltpu.SemaphoreType.DMA,
        pltpu.SemaphoreType.DMA,
    ],
)
def _sc_kernel(mask_hbm, table_hbm, out_hbm, mask_v, tbuf, obuf, sem_t, sem_o):
    _sc_body(mask_hbm, table_hbm, out_hbm, mask_v, tbuf, obuf, sem_t, sem_o)


def kernel(x, mask, table):
    del x
    return _sc_kernel(mask, table[:_S])


# speedup vs baseline: 1.3815x; 1.3815x over previous
import functools

import jax
import jax.numpy as jnp
from jax import lax
from jax.experimental import pallas as pl
from jax.experimental.pallas import tpu as pltpu
from jax.experimental.pallas import tpu_sc as plsc

_NC, _NS, _L = 2, 16, 16
_NW = _NC * _NS
_B = 4
_S = 8192
_D = 1024
_ROWS = _S // _NW
_CH = 8
_NCH = _ROWS // _CH


def _sc_body(mask_hbm, table_hbm, out_hbm, mask_v, tbuf, obuf, sem_t, sem_o):
    wid = lax.axis_index("s") * _NC + lax.axis_index("c")
    base = wid * _ROWS

    for b in range(_B):
        pltpu.sync_copy(
            mask_hbm.at[b, pl.ds(base, _ROWS)], mask_v.at[pl.ds(b * _ROWS, _ROWS)]
        )

    pltpu.async_copy(table_hbm.at[pl.ds(base, _CH)], tbuf.at[0], sem_t)

    def compute_chunk(c, k):
        crow = c * _CH

        nxt = jnp.minimum(c + 1, _NCH - 1) * _CH
        pltpu.async_copy(table_hbm.at[pl.ds(base + nxt, _CH)], tbuf.at[1 - k], sem_t)

        pltpu.make_async_copy(
            table_hbm.at[pl.ds(base + crow, _CH)], tbuf.at[k], sem_t
        ).wait()

        @pl.when(c >= 2)
        def _():
            pltpu.make_async_copy(
                obuf.at[k], out_hbm.at[:, pl.ds(base, _CH)], sem_o
            ).wait()

        @plsc.parallel_loop(0, _CH)
        def _row(r):
            mb = [
                jnp.broadcast_to(
                    mask_v[pl.ds(b * _ROWS + crow + r, _L)][0], (_L,)
                )
                for b in range(_B)
            ]
            for dch in range(_D // _L):
                tv = tbuf[k, r, pl.ds(dch * _L, _L)]
                for b in range(_B):
                    obuf[k, b, r, pl.ds(dch * _L, _L)] = tv * mb[b]

        pltpu.async_copy(
            obuf.at[k], out_hbm.at[:, pl.ds(base + crow, _CH)], sem_o
        )

    def outer(g, carry):
        for k in range(2):
            compute_chunk(g * 2 + k, k)
        return carry

    lax.fori_loop(0, _NCH // 2, outer, 0, unroll=False)

    pltpu.make_async_copy(
        table_hbm.at[pl.ds(base, _CH)], tbuf.at[1], sem_t
    ).wait()
    for k in range(2):
        pltpu.make_async_copy(
            obuf.at[k], out_hbm.at[:, pl.ds(base, _CH)], sem_o
        ).wait()


@functools.partial(
    pl.kernel,
    out_type=jax.ShapeDtypeStruct((_B, _S, _D), jnp.float32),
    mesh=plsc.VectorSubcoreMesh(core_axis_name="c", subcore_axis_name="s"),
    scratch_types=[
        pltpu.VMEM((_B * _ROWS + _L,), jnp.float32),
        pltpu.VMEM((2, _CH, _D), jnp.float32),
        pltpu.VMEM((2, _B, _CH, _D), jnp.float32),
        pltpu.SemaphoreType.DMA,
        pltpu.SemaphoreType.DMA,
    ],
)
def _sc_kernel(mask_hbm, table_hbm, out_hbm, mask_v, tbuf, obuf, sem_t, sem_o):
    _sc_body(mask_hbm, table_hbm, out_hbm, mask_v, tbuf, obuf, sem_t, sem_o)


def kernel(x, mask, table):
    del x
    return _sc_kernel(mask, table[:_S])


# speedup vs baseline: 1.4193x; 1.0274x over previous
import functools

import jax
import jax.numpy as jnp
from jax import lax
from jax.experimental import pallas as pl
from jax.experimental.pallas import tpu as pltpu
from jax.experimental.pallas import tpu_sc as plsc

_NC, _NS, _L = 2, 16, 16
_NW = _NC * _NS
_B = 4
_S = 8192
_D = 1024
_ROWS = _S // _NW
_CH = 8
_NCH = _ROWS // _CH


def _sc_body(mask_hbm, table_hbm, out_hbm, mask_v, tbuf, obuf, sem_t, sem_o):
    wid = lax.axis_index("s") * _NC + lax.axis_index("c")
    base = wid * _ROWS

    pltpu.async_copy(table_hbm.at[pl.ds(base, _CH)], tbuf.at[0], sem_t)
    for b in range(_B):
        pltpu.async_copy(
            mask_hbm.at[b, pl.ds(base, _ROWS)], mask_v.at[pl.ds(b * _ROWS, _ROWS)],
            sem_o,
        )
    for b in range(_B):
        pltpu.make_async_copy(
            mask_hbm.at[b, pl.ds(base, _ROWS)], mask_v.at[pl.ds(b * _ROWS, _ROWS)],
            sem_o,
        ).wait()

    def compute_chunk(c, k):
        crow = c * _CH

        nxt = jnp.minimum(c + 1, _NCH - 1) * _CH
        pltpu.async_copy(table_hbm.at[pl.ds(base + nxt, _CH)], tbuf.at[1 - k], sem_t)

        @pl.when(c >= 2)
        def _():
            pltpu.make_async_copy(
                obuf.at[k], out_hbm.at[:, pl.ds(base, _CH)], sem_o
            ).wait()

        pltpu.make_async_copy(
            table_hbm.at[pl.ds(base + crow, _CH)], tbuf.at[k], sem_t
        ).wait()

        @plsc.parallel_loop(0, _CH)
        def _row(r):
            mb = [
                jnp.broadcast_to(
                    mask_v[pl.ds(b * _ROWS + crow + r, _L)][0], (_L,)
                )
                for b in range(_B)
            ]
            for dch in range(_D // _L):
                tv = tbuf[k, r, pl.ds(dch * _L, _L)]
                for b in range(_B):
                    obuf[k, b, r, pl.ds(dch * _L, _L)] = tv * mb[b]

        pltpu.async_copy(
            obuf.at[k], out_hbm.at[:, pl.ds(base + crow, _CH)], sem_o
        )

    def outer(g, carry):
        for k in range(2):
            compute_chunk(g * 2 + k, k)
        return carry

    lax.fori_loop(0, _NCH // 2, outer, 0, unroll=False)

    pltpu.make_async_copy(
        table_hbm.at[pl.ds(base, _CH)], tbuf.at[1], sem_t
    ).wait()
    for k in range(2):
        pltpu.make_async_copy(
            obuf.at[k], out_hbm.at[:, pl.ds(base, _CH)], sem_o
        ).wait()


@functools.partial(
    pl.kernel,
    out_type=jax.ShapeDtypeStruct((_B, _S, _D), jnp.float32),
    mesh=plsc.VectorSubcoreMesh(core_axis_name="c", subcore_axis_name="s"),
    scratch_types=[
        pltpu.VMEM((_B * _ROWS + _L,), jnp.float32),
        pltpu.VMEM((2, _CH, _D), jnp.float32),
        pltpu.VMEM((2, _B, _CH, _D), jnp.float32),
        pltpu.SemaphoreType.DMA,
        pltpu.SemaphoreType.DMA,
    ],
)
def _sc_kernel(mask_hbm, table_hbm, out_hbm, mask_v, tbuf, obuf, sem_t, sem_o):
    _sc_body(mask_hbm, table_hbm, out_hbm, mask_v, tbuf, obuf, sem_t, sem_o)


def kernel(x, mask, table):
    del x
    return _sc_kernel(mask, table[:_S])
